# Initial kernel scaffold; baseline (speedup 1.0000x reference)
import jax
import jax.numpy as jnp
from jax.experimental import pallas as pl


def kernel(x, batch_index, W1, b1, W2, b2):
    raise NotImplementedError("write your pallas kernel here")


# speedup vs baseline: 5.0491x; 5.0491x over previous
import functools

import jax
import jax.numpy as jnp
from jax import lax
from jax.experimental import pallas as pl
from jax.experimental.pallas import tpu as pltpu
from jax.experimental.pallas import tpu_sc as plsc

_NUM_SEG = 512
_NC = 2
_NS = 16
_NW = _NC * _NS
_LANES = 16
_HIST_W = 528


def _sc_hist_body(chunk, idx_hbm, out_hbm, idx_v, hist_v):
    wid = lax.axis_index("s") * _NC + lax.axis_index("c")
    base = wid * chunk
    pltpu.sync_copy(idx_hbm.at[pl.ds(base, chunk)], idx_v)

    zeros = jnp.zeros((_LANES,), jnp.float32)

    def zero_body(i, c):
        hist_v[pl.ds(i * _LANES, _LANES)] = zeros
        return c

    lax.fori_loop(0, _HIST_W // _LANES, zero_body, 0)

    ones = jnp.ones((_LANES,), jnp.float32)

    def acc_body(i, c):
        iv = idx_v[pl.ds(i * _LANES, _LANES)]
        plsc.addupdate_scatter(hist_v, [iv], ones)
        return c

    lax.fori_loop(0, chunk // _LANES, acc_body, 0)
    pltpu.sync_copy(hist_v, out_hbm.at[wid])


@functools.lru_cache(maxsize=None)
def _make_sc_hist(n_pad):
    chunk = n_pad // _NW
    mesh = plsc.VectorSubcoreMesh(core_axis_name="c", subcore_axis_name="s")
    return pl.kernel(
        functools.partial(_sc_hist_body, chunk),
        mesh=mesh,
        out_type=jax.ShapeDtypeStruct((_NW, _HIST_W), jnp.float32),
        scratch_types=[
            pltpu.VMEM((chunk,), jnp.int32),
            pltpu.VMEM((_HIST_W,), jnp.float32),
        ],
        compiler_params=pltpu.CompilerParams(needs_layout_passes=False),
    )


def _tc_mlp_body(parts_ref, w1_ref, b1_ref, w2_ref, b2_ref, out_ref):
    counts = jnp.sum(parts_ref[...], axis=0)
    inv = (1.0 / counts).reshape(_NUM_SEG, 1)
    h = jnp.maximum(inv * w1_ref[...] + b1_ref[...], 0.0)
    out_ref[...] = (
        jnp.dot(h, w2_ref[...], preferred_element_type=jnp.float32)
        + b2_ref[...]
    )


def kernel(x, batch_index, W1, b1, W2, b2):
    del x
    idx = batch_index.astype(jnp.int32)
    n = idx.shape[0]
    n_pad = n + (-n) % (_NW * _LANES)
    if n_pad != n:
        idx = jnp.concatenate(
            [idx, jnp.full((n_pad - n,), _NUM_SEG, jnp.int32)]
        )
    parts = _make_sc_hist(n_pad)(idx)[:, :_NUM_SEG]
    out = pl.pallas_call(
        _tc_mlp_body,
        out_shape=jax.ShapeDtypeStruct((_NUM_SEG, 128), jnp.float32),
    )(parts, W1, b1.reshape(1, 128), W2, b2.reshape(1, 128))
    return out


# speedup vs baseline: 5.3897x; 1.0675x over previous
import functools

import jax
import jax.numpy as jnp
from jax import lax
from jax.experimental import pallas as pl
from jax.experimental.pallas import tpu as pltpu
from jax.experimental.pallas import tpu_sc as plsc

_NUM_SEG = 512
_NC = 2
_NS = 16
_NW = _NC * _NS
_LANES = 16
_HIST_W = 528


def _sc_hist_body(n, chunk, idx_hbm, out_hbm, idx_v, hist_v):
    wid = lax.axis_index("s") * _NC + lax.axis_index("c")
    base = wid * chunk
    last_len = n - (_NW - 1) * chunk

    @pl.when(wid < _NW - 1)
    def _():
        pltpu.sync_copy(idx_hbm.at[pl.ds(base, chunk)], idx_v)

    @pl.when(wid == _NW - 1)
    def _():
        pltpu.sync_copy(
            idx_hbm.at[pl.ds(base, last_len)], idx_v.at[pl.ds(0, last_len)]
        )

    zeros = jnp.zeros((_LANES,), jnp.float32)

    def zero_body(i, c):
        hist_v[pl.ds(i * _LANES, _LANES)] = zeros
        return c

    lax.fori_loop(0, _HIST_W // _LANES, zero_body, 0)

    ones = jnp.ones((_LANES,), jnp.float32)

    def acc_body(i, c):
        iv = idx_v[pl.ds(i * _LANES, _LANES)]
        plsc.addupdate_scatter(hist_v, [iv], ones)
        return c

    nv = jnp.where(wid == _NW - 1, last_len // _LANES, chunk // _LANES)
    lax.fori_loop(0, nv, acc_body, 0)
    pltpu.sync_copy(hist_v, out_hbm.at[wid])


@functools.lru_cache(maxsize=None)
def _make_sc_hist(n):
    chunk = -(-n // _NW)
    chunk += (-chunk) % _LANES
    mesh = plsc.VectorSubcoreMesh(core_axis_name="c", subcore_axis_name="s")
    return pl.kernel(
        functools.partial(_sc_hist_body, n, chunk),
        mesh=mesh,
        out_type=jax.ShapeDtypeStruct((_NW, _HIST_W), jnp.float32),
        scratch_types=[
            pltpu.VMEM((chunk,), jnp.int32),
            pltpu.VMEM((_HIST_W,), jnp.float32),
        ],
        compiler_params=pltpu.CompilerParams(needs_layout_passes=False),
    )


def _tc_mlp_body(parts_ref, w1_ref, b1_ref, w2_ref, b2_ref, out_ref):
    counts = jnp.sum(parts_ref[:, :_NUM_SEG], axis=0)
    inv = (1.0 / counts).reshape(_NUM_SEG, 1)
    h = jnp.maximum(inv * w1_ref[...] + b1_ref[...], 0.0)
    out_ref[...] = (
        jnp.dot(h, w2_ref[...], preferred_element_type=jnp.float32)
        + b2_ref[...]
    )


def kernel(x, batch_index, W1, b1, W2, b2):
    del x
    idx = batch_index.astype(jnp.int32)
    n = idx.shape[0]
    if n % _LANES != 0:
        pad = (-n) % _LANES
        idx = jnp.concatenate([idx, jnp.full((pad,), _NUM_SEG, jnp.int32)])
        n += pad
    parts = _make_sc_hist(n)(idx)
    out = pl.pallas_call(
        _tc_mlp_body,
        out_shape=jax.ShapeDtypeStruct((_NUM_SEG, 128), jnp.float32),
    )(parts, W1, b1.reshape(1, 128), W2, b2.reshape(1, 128))
    return out


# speedup vs baseline: 5.8260x; 1.0809x over previous
import functools

import jax
import jax.numpy as jnp
from jax import lax
from jax.experimental import pallas as pl
from jax.experimental.pallas import tpu as pltpu
from jax.experimental.pallas import tpu_sc as plsc

_NUM_SEG = 512
_NC = 2
_NS = 16
_NW = _NC * _NS
_LANES = 16
_HALF = 528
_TAB_W = 2 * _HALF


def _sc_bounds_body(n, chunk, idx_hbm, out_hbm, idx_v, tab_v, sem):
    wid = lax.axis_index("s") * _NC + lax.axis_index("c")
    base = wid * chunk
    last_len = n - (_NW - 1) * chunk
    is_tail = wid == _NW - 1

    @pl.when(wid == 0)
    def _():
        idx_v[pl.ds(0, _LANES)] = jnp.full((_LANES,), -1, jnp.int32)
        pltpu.async_copy(
            idx_hbm.at[pl.ds(0, chunk + _LANES)],
            idx_v.at[pl.ds(_LANES, chunk + _LANES)],
            sem,
        )

    @pl.when(jnp.logical_and(wid > 0, jnp.logical_not(is_tail)))
    def _():
        pltpu.async_copy(
            idx_hbm.at[pl.ds(base - _LANES, chunk + 2 * _LANES)],
            idx_v.at[pl.ds(0, chunk + 2 * _LANES)],
            sem,
        )

    @pl.when(is_tail)
    def _():
        pltpu.async_copy(
            idx_hbm.at[pl.ds(base - _LANES, last_len + _LANES)],
            idx_v.at[pl.ds(0, last_len + _LANES)],
            sem,
        )

    zeros = jnp.zeros((_LANES,), jnp.float32)

    def zero_body(i, c):
        tab_v[pl.ds(i * _LANES, _LANES)] = zeros
        return c

    lax.fori_loop(0, _TAB_W // _LANES, zero_body, 0)

    @pl.when(wid == 0)
    def _():
        pltpu.make_async_copy(
            idx_hbm.at[pl.ds(0, chunk + _LANES)],
            idx_v.at[pl.ds(_LANES, chunk + _LANES)],
            sem,
        ).wait()

    @pl.when(jnp.logical_and(wid > 0, jnp.logical_not(is_tail)))
    def _():
        pltpu.make_async_copy(
            idx_hbm.at[pl.ds(base - _LANES, chunk + 2 * _LANES)],
            idx_v.at[pl.ds(0, chunk + 2 * _LANES)],
            sem,
        ).wait()

    @pl.when(is_tail)
    def _():
        pltpu.make_async_copy(
            idx_hbm.at[pl.ds(base - _LANES, last_len + _LANES)],
            idx_v.at[pl.ds(0, last_len + _LANES)],
            sem,
        ).wait()
        idx_v[pl.ds(_LANES + last_len, _LANES)] = jnp.full(
            (_LANES,), -1, jnp.int32
        )

    lane = lax.iota(jnp.int32, _LANES)
    half = jnp.full((_LANES,), _HALF, jnp.int32)

    def acc_body(i, c):
        v = idx_v[pl.ds(_LANES + i * _LANES, _LANES)]
        p = idx_v[pl.ds(_LANES - 1 + i * _LANES, _LANES)]
        q = idx_v[pl.ds(_LANES + 1 + i * _LANES, _LANES)]
        pos = ((base + i * _LANES) + lane).astype(jnp.float32)
        plsc.store_scatter(tab_v, [v], pos, mask=v != p)
        plsc.store_scatter(tab_v, [v + half], pos + 1.0, mask=v != q)
        return c

    nv = jnp.where(is_tail, last_len // _LANES, chunk // _LANES)
    lax.fori_loop(0, nv, acc_body, 0)
    pltpu.sync_copy(tab_v, out_hbm.at[wid])


@functools.lru_cache(maxsize=None)
def _make_sc_bounds(n):
    chunk = -(-n // _NW)
    chunk += (-chunk) % _LANES
    mesh = plsc.VectorSubcoreMesh(core_axis_name="c", subcore_axis_name="s")
    return pl.kernel(
        functools.partial(_sc_bounds_body, n, chunk),
        mesh=mesh,
        out_type=jax.ShapeDtypeStruct((_NW, _TAB_W), jnp.float32),
        scratch_types=[
            pltpu.VMEM((chunk + 2 * _LANES,), jnp.int32),
            pltpu.VMEM((_TAB_W,), jnp.float32),
            pltpu.SemaphoreType.DMA,
        ],
        compiler_params=pltpu.CompilerParams(needs_layout_passes=False),
    )


def _tc_mlp_body(parts_ref, w1_ref, b1_ref, w2_ref, b2_ref, out_ref):
    tab = jnp.sum(parts_ref[...], axis=0)
    counts = tab[_HALF : _HALF + _NUM_SEG] - tab[:_NUM_SEG]
    inv = (1.0 / counts).reshape(_NUM_SEG, 1)
    h = jnp.maximum(inv * w1_ref[...] + b1_ref[...], 0.0)
    out_ref[...] = (
        jnp.dot(h, w2_ref[...], preferred_element_type=jnp.float32)
        + b2_ref[...]
    )


def kernel(x, batch_index, W1, b1, W2, b2):
    del x
    idx = batch_index.astype(jnp.int32)
    n = idx.shape[0]
    if n % _LANES != 0:
        pad = (-n) % _LANES
        idx = jnp.concatenate([idx, jnp.full((pad,), _NUM_SEG, jnp.int32)])
        n += pad
    parts = _make_sc_bounds(n)(idx)
    out = pl.pallas_call(
        _tc_mlp_body,
        out_shape=jax.ShapeDtypeStruct((_NUM_SEG, 128), jnp.float32),
    )(parts, W1, b1.reshape(1, 128), W2, b2.reshape(1, 128))
    return out


# speedup vs baseline: 6.2283x; 1.0690x over previous
import functools

import jax
import jax.numpy as jnp
from jax import lax
from jax.experimental import pallas as pl
from jax.experimental.pallas import tpu as pltpu
from jax.experimental.pallas import tpu_sc as plsc

_NUM_SEG = 512
_NC = 2
_NS = 16
_NW = _NC * _NS
_LANES = 16
_HALF = 528
_TAB_W = 2 * _HALF


def _sc_bounds_body(n, chunk, idx_hbm, out_hbm, idx_v, tab_v, sem):
    wid = lax.axis_index("s") * _NC + lax.axis_index("c")
    base = wid * chunk
    last_len = n - (_NW - 1) * chunk
    is_tail = wid == _NW - 1

    @pl.when(wid == 0)
    def _():
        idx_v[pl.ds(0, _LANES)] = jnp.full((_LANES,), -1, jnp.int32)
        pltpu.async_copy(
            idx_hbm.at[pl.ds(0, chunk + _LANES)],
            idx_v.at[pl.ds(_LANES, chunk + _LANES)],
            sem,
        )

    @pl.when(jnp.logical_and(wid > 0, jnp.logical_not(is_tail)))
    def _():
        pltpu.async_copy(
            idx_hbm.at[pl.ds(base - _LANES, chunk + 2 * _LANES)],
            idx_v.at[pl.ds(0, chunk + 2 * _LANES)],
            sem,
        )

    @pl.when(is_tail)
    def _():
        pltpu.async_copy(
            idx_hbm.at[pl.ds(base - _LANES, last_len + _LANES)],
            idx_v.at[pl.ds(0, last_len + _LANES)],
            sem,
        )

    zeros = jnp.zeros((_LANES,), jnp.float32)

    @plsc.parallel_loop(0, _TAB_W // _LANES, 1, unroll=4)
    def _(i):
        tab_v[pl.ds(i * _LANES, _LANES)] = zeros

    @pl.when(wid == 0)
    def _():
        pltpu.make_async_copy(
            idx_hbm.at[pl.ds(0, chunk + _LANES)],
            idx_v.at[pl.ds(_LANES, chunk + _LANES)],
            sem,
        ).wait()

    @pl.when(jnp.logical_and(wid > 0, jnp.logical_not(is_tail)))
    def _():
        pltpu.make_async_copy(
            idx_hbm.at[pl.ds(base - _LANES, chunk + 2 * _LANES)],
            idx_v.at[pl.ds(0, chunk + 2 * _LANES)],
            sem,
        ).wait()

    @pl.when(is_tail)
    def _():
        pltpu.make_async_copy(
            idx_hbm.at[pl.ds(base - _LANES, last_len + _LANES)],
            idx_v.at[pl.ds(0, last_len + _LANES)],
            sem,
        ).wait()
        idx_v[pl.ds(_LANES + last_len, _LANES)] = jnp.full(
            (_LANES,), -1, jnp.int32
        )

    lane = lax.iota(jnp.int32, _LANES)
    half = jnp.full((_LANES,), _HALF, jnp.int32)

    nv = jnp.where(is_tail, last_len // _LANES, chunk // _LANES)

    @plsc.parallel_loop(0, nv, 1, unroll=4)
    def _(i):
        v = idx_v[pl.ds(_LANES + i * _LANES, _LANES)]
        p = idx_v[pl.ds(_LANES - 1 + i * _LANES, _LANES)]
        q = idx_v[pl.ds(_LANES + 1 + i * _LANES, _LANES)]
        pos = ((base + i * _LANES) + lane).astype(jnp.float32)
        plsc.store_scatter(tab_v, [v], pos, mask=v != p)
        plsc.store_scatter(tab_v, [v + half], pos + 1.0, mask=v != q)
    pltpu.sync_copy(tab_v, out_hbm.at[wid])


@functools.lru_cache(maxsize=None)
def _make_sc_bounds(n):
    chunk = -(-n // _NW)
    chunk += (-chunk) % _LANES
    mesh = plsc.VectorSubcoreMesh(core_axis_name="c", subcore_axis_name="s")
    return pl.kernel(
        functools.partial(_sc_bounds_body, n, chunk),
        mesh=mesh,
        out_type=jax.ShapeDtypeStruct((_NW, _TAB_W), jnp.float32),
        scratch_types=[
            pltpu.VMEM((chunk + 2 * _LANES,), jnp.int32),
            pltpu.VMEM((_TAB_W,), jnp.float32),
            pltpu.SemaphoreType.DMA,
        ],
        compiler_params=pltpu.CompilerParams(needs_layout_passes=False),
    )


def _tc_mlp_body(parts_ref, w1_ref, b1_ref, w2_ref, b2_ref, out_ref):
    tab = jnp.sum(parts_ref[...], axis=0)
    counts = tab[_HALF : _HALF + _NUM_SEG] - tab[:_NUM_SEG]
    inv = (1.0 / counts).reshape(_NUM_SEG, 1)
    h = jnp.maximum(inv * w1_ref[...] + b1_ref[...], 0.0)
    out_ref[...] = (
        jnp.dot(h, w2_ref[...], preferred_element_type=jnp.float32)
        + b2_ref[...]
    )


def kernel(x, batch_index, W1, b1, W2, b2):
    del x
    idx = batch_index.astype(jnp.int32)
    n = idx.shape[0]
    if n % _LANES != 0:
        pad = (-n) % _LANES
        idx = jnp.concatenate([idx, jnp.full((pad,), _NUM_SEG, jnp.int32)])
        n += pad
    parts = _make_sc_bounds(n)(idx)
    out = pl.pallas_call(
        _tc_mlp_body,
        out_shape=jax.ShapeDtypeStruct((_NUM_SEG, 128), jnp.float32),
    )(parts, W1, b1.reshape(1, 128), W2, b2.reshape(1, 128))
    return out
